# initial kernel scaffold (unmeasured)
import jax
import jax.numpy as jnp
from jax import lax
from jax.experimental import pallas as pl
from jax.experimental.pallas import tpu as pltpu


def kernel(
    x,
):
    def body(*refs):
        pass

    out_shape = jax.ShapeDtypeStruct(..., jnp.float32)
    return pl.pallas_call(body, out_shape=out_shape)(...)


# baseline (device time: 6924 ns/iter reference)
import jax
import jax.numpy as jnp
from jax import lax
from jax.experimental import pallas as pl
from jax.experimental.pallas import tpu as pltpu

N_DEV = 4
ROWS = 8


def kernel(x):
    m, n = x.shape

    def body(x_ref, out_ref, tot_ref, own_ref, send_sems, recv_sems):
        my = lax.axis_index("i")

        tot_ref[...] = jnp.zeros_like(tot_ref)

        rows = lax.broadcasted_iota(jnp.int32, (m, m), 0)
        cols = lax.broadcasted_iota(jnp.int32, (m, m), 1)
        tril = (rows >= cols).astype(jnp.bfloat16)
        xb = x_ref[...].astype(jnp.bfloat16)
        cum = lax.dot_general(
            tril,
            xb,
            (((1,), (0,)), ((), ())),
            preferred_element_type=jnp.float32,
        )
        own_ref[...] = jnp.broadcast_to(cum[-1:, :], own_ref.shape)

        barrier = pltpu.get_barrier_semaphore()
        for d in range(N_DEV):

            @pl.when(my != d)
            def _():
                pl.semaphore_signal(
                    barrier,
                    inc=1,
                    device_id=(d,),
                    device_id_type=pl.DeviceIdType.MESH,
                )

        pl.semaphore_wait(barrier, N_DEV - 1)

        def xfer(k, j):
            return pltpu.make_async_remote_copy(
                src_ref=own_ref,
                dst_ref=tot_ref.at[k],
                send_sem=send_sems.at[j],
                recv_sem=recv_sems.at[k],
                device_id=(j,),
                device_id_type=pl.DeviceIdType.MESH,
            )

        for k in range(N_DEV):
            for j in range(k + 1, N_DEV):

                @pl.when(my == k)
                def _():
                    xfer(k, j).start()

        for k in range(N_DEV):
            for j in range(k + 1, N_DEV):

                @pl.when(my == j)
                def _():
                    xfer(k, j).wait_recv()

        offset = tot_ref[0, 0:1, :] + tot_ref[1, 0:1, :] + tot_ref[2, 0:1, :]
        out_ref[...] = cum + offset

        for k in range(N_DEV):
            for j in range(k + 1, N_DEV):

                @pl.when(my == k)
                def _():
                    xfer(k, j).wait_send()

    return pl.pallas_call(
        body,
        out_shape=jax.ShapeDtypeStruct((m, n), x.dtype),
        in_specs=[pl.BlockSpec(memory_space=pltpu.VMEM)],
        out_specs=pl.BlockSpec(memory_space=pltpu.VMEM),
        scratch_shapes=[
            pltpu.VMEM((N_DEV, ROWS, n), jnp.float32),
            pltpu.VMEM((ROWS, n), jnp.float32),
            pltpu.SemaphoreType.DMA((N_DEV,)),
            pltpu.SemaphoreType.DMA((N_DEV,)),
        ],
        compiler_params=pltpu.CompilerParams(collective_id=0),
    )(x)
